# baseline (speedup 1.0000x reference)
import dataclasses
import functools

import jax
import jax.numpy as jnp
import numpy as np
from jax import lax
from jax.experimental import pallas as pl
from jax.experimental.pallas import tpu as pltpu
from jax.experimental.pallas import tpu_sc as plsc

D = 128
NTOK = 320000
NNODES = 10000
NPAD = 10240

_STATS_R = 8000
_MLP_R = 4000
_NS = 16
_NW = 32
_CHUNKS = (320000,)

_SC_CHUNK = 128


def _stats_body(x_ref, s_ref, q_ref):
    i = pl.program_id(0)
    xb = x_ref[...]

    @pl.when(i == 0)
    def _():
        s_ref[...] = jnp.zeros_like(s_ref)
        q_ref[...] = jnp.zeros_like(q_ref)

    s_ref[...] += jnp.sum(xb, axis=0, keepdims=True)
    q_ref[...] += jnp.sum(xb * xb, axis=0, keepdims=True)


def _stats(x):
    return pl.pallas_call(
        _stats_body,
        grid=(NTOK // _STATS_R,),
        in_specs=[pl.BlockSpec((_STATS_R, D), lambda i: (i, 0))],
        out_specs=(pl.BlockSpec((1, D), lambda i: (0, 0)),
                   pl.BlockSpec((1, D), lambda i: (0, 0))),
        out_shape=(jax.ShapeDtypeStruct((1, D), jnp.float32),
                   jax.ShapeDtypeStruct((1, D), jnp.float32)),
    )(x)


def _mlp_body(s_ref, q_ref, bnw_ref, bnb_ref, w1_ref, b1_ref, w2_ref, b2_ref,
              x_ref, h_ref):
    mu = s_ref[...] / NTOK
    var = q_ref[...] / NTOK - mu * mu
    scale = bnw_ref[...] * lax.rsqrt(var + 1e-5)
    shift = bnb_ref[...] - mu * scale
    xn = (x_ref[...] * scale + shift).astype(jnp.bfloat16)
    g = lax.dot_general(xn, w1_ref[...], (((1,), (1,)), ((), ())),
                        preferred_element_type=jnp.float32)
    g = jnp.maximum(g + b1_ref[...], 0.0).astype(jnp.bfloat16)
    h = lax.dot_general(g, w2_ref[...], (((1,), (1,)), ((), ())),
                        preferred_element_type=jnp.float32)
    h = h + b2_ref[...]
    h_ref[0, :, :] = h[:, :D]
    h_ref[1, :, :] = h[:, D:]


def _mlp(x, tok0, chtok, s, q, bn1_w, bn1_b, w1, b1, w2, b2):
    full = lambda shape: pl.BlockSpec(shape, lambda i: tuple(0 for _ in shape))
    base = tok0 // _MLP_R
    return pl.pallas_call(
        _mlp_body,
        grid=(chtok // _MLP_R,),
        in_specs=[full((1, D)), full((1, D)), full((1, D)), full((1, D)),
                  full((2 * D, D)), full((1, 2 * D)),
                  full((2 * D, 2 * D)), full((1, 2 * D)),
                  pl.BlockSpec((_MLP_R, D), lambda i: (base + i, 0))],
        out_specs=pl.BlockSpec((2, _MLP_R, D), lambda i: (0, i, 0)),
        out_shape=jax.ShapeDtypeStruct((2, chtok, D), jnp.float32),
    )(s, q, bn1_w.reshape(1, D), bn1_b.reshape(1, D),
      w1.astype(jnp.bfloat16), b1.reshape(1, 2 * D),
      w2.astype(jnp.bfloat16), b2.reshape(1, 2 * D), x)


def _segment_sum(h_rows, idx, init, real_ch, w_ch):
    pad_ch = w_ch * _NW
    npair = w_ch // 2
    mesh = plsc.VectorSubcoreMesh(core_axis_name="c", subcore_axis_name="s")
    cp = pltpu.CompilerParams()
    if "needs_layout_passes" in pltpu.CompilerParams.__dataclass_fields__:
        cp = dataclasses.replace(cp, needs_layout_passes=False)

    @functools.partial(
        pl.kernel,
        mesh=mesh,
        compiler_params=cp,
        out_type=jax.ShapeDtypeStruct((2, NPAD, D), jnp.float32),
        scratch_types=[
            pltpu.VMEM_SHARED((NPAD, D), jnp.float32),
            pltpu.VMEM((_SC_CHUNK, D), jnp.float32),
            pltpu.VMEM((_SC_CHUNK, D), jnp.float32),
            pltpu.VMEM((1, _SC_CHUNK), jnp.int32),
            pltpu.VMEM((1, _SC_CHUNK), jnp.int32),
            pltpu.SemaphoreType.DMA,
            pltpu.SemaphoreType.DMA,
            pltpu.SemaphoreType.DMA,
            pltpu.SemaphoreType.DMA,
        ],
    )
    def scatter_kernel(h_hbm, idx_hbm, init_hbm, out_hbm, acc,
                       buf_a, buf_b, ib_a, ib_b,
                       ra, ia, sa, sb):
        core = lax.axis_index("c")
        sid = lax.axis_index("s")
        wid = sid * 2 + core
        rows_per_tile = NPAD // _NS
        sl = pl.ds(sid * rows_per_tile, rows_per_tile)
        pltpu.sync_copy(init_hbm.at[core].at[sl], acc.at[sl])
        plsc.subcore_barrier()

        def rows_at(c):
            cr = jnp.minimum(c, real_ch - 1)
            return h_hbm.at[pl.ds(cr * _SC_CHUNK, _SC_CHUNK)]

        def start_in(c, buf, ibuf):
            cc = jnp.minimum(c, pad_ch - 1)
            pltpu.async_copy(rows_at(cc), buf, ra)
            pltpu.async_copy(idx_hbm.at[cc], ibuf, ia)

        def wait_in(buf, ibuf):
            pltpu.make_async_copy(rows_at(0), buf, ra).wait()
            pltpu.make_async_copy(idx_hbm.at[0], ibuf, ia).wait()

        def start_scatter(buf, ibuf, sem):
            pltpu.async_copy(buf, acc.at[ibuf.at[0]], sem, add=True)

        def wait_scatter(buf, ibuf, sem):
            pltpu.make_async_copy(buf, acc.at[ibuf.at[0]], sem).wait()

        base = wid * w_ch
        start_in(base, buf_a, ib_a)

        @pl.loop(0, npair)
        def _(jp):
            c = base + 2 * jp
            wait_in(buf_a, ib_a)

            @pl.when(jp > 0)
            def _():
                wait_scatter(buf_b, ib_b, sb)

            start_in(c + 1, buf_b, ib_b)
            start_scatter(buf_a, ib_a, sa)
            wait_in(buf_b, ib_b)
            wait_scatter(buf_a, ib_a, sa)
            start_in(c + 2, buf_a, ib_a)
            start_scatter(buf_b, ib_b, sb)

        wait_scatter(buf_b, ib_b, sb)
        wait_in(buf_a, ib_a)
        plsc.subcore_barrier()
        pltpu.sync_copy(acc.at[sl], out_hbm.at[core].at[sl])

    return scatter_kernel(h_rows, idx, init)


def _node_body(p_ref, bnw_ref, bnb_ref, w3_ref, b3_ref, w4_ref, b4_ref, o_ref):
    nf = p_ref[0, :NNODES, :] + p_ref[1, :NNODES, :]
    mu = jnp.mean(nf, axis=0, keepdims=True)
    var = jnp.mean(nf * nf, axis=0, keepdims=True) - mu * mu
    xn = ((nf - mu) * lax.rsqrt(var + 1e-5) * bnw_ref[...]
          + bnb_ref[...]).astype(jnp.bfloat16)
    g = lax.dot_general(xn, w3_ref[...], (((1,), (1,)), ((), ())),
                        preferred_element_type=jnp.float32)
    g = jnp.maximum(g + b3_ref[...], 0.0).astype(jnp.bfloat16)
    o = lax.dot_general(g, w4_ref[...], (((1,), (1,)), ((), ())),
                        preferred_element_type=jnp.float32)
    o_ref[...] = o + b4_ref[...]


def _node_mlp(partial, bn2_w, bn2_b, w3, b3, w4, b4):
    return pl.pallas_call(
        _node_body,
        out_shape=jax.ShapeDtypeStruct((NNODES, D), jnp.float32),
    )(partial, bn2_w.reshape(1, D), bn2_b.reshape(1, D),
      w3.astype(jnp.bfloat16), b3.reshape(1, D),
      w4.astype(jnp.bfloat16), b4.reshape(1, D))


def kernel(x, node_features, node_batch, token_index, bn1_w, bn1_b, w1, b1,
           w2, b2, bn2_w, bn2_b, w3, b3, w4, b4):
    s, q = _stats(x)
    acc = jnp.zeros((2, NPAD, D), jnp.float32)
    tok0 = 0
    for chtok in _CHUNKS:
        h = _mlp(x, tok0, chtok, s, q, bn1_w, bn1_b, w1, b1, w2, b2)
        ch_rows = 2 * chtok
        real_ch = ch_rows // _SC_CHUNK
        w_ch = ((real_ch + _NW - 1) // _NW + 1) // 2 * 2
        pad_ch = w_ch * _NW
        pad_idx = jnp.full((pad_ch * _SC_CHUNK - ch_rows,), NNODES, jnp.int32)
        idx = jnp.concatenate(
            [token_index[:, tok0:tok0 + chtok].reshape(-1), pad_idx]
        ).reshape(pad_ch, 1, _SC_CHUNK)
        acc = _segment_sum(h.reshape(ch_rows, D), idx, acc, real_ch, w_ch)
        tok0 += chtok
    return _node_mlp(acc, bn2_w, bn2_b, w3, b3, w4, b4)


# speedup vs baseline: 1.0650x; 1.0650x over previous
import dataclasses
import functools

import jax
import jax.numpy as jnp
import numpy as np
from jax import lax
from jax.experimental import pallas as pl
from jax.experimental.pallas import tpu as pltpu
from jax.experimental.pallas import tpu_sc as plsc

D = 128
NTOK = 320000
NNODES = 10000
NPAD = 10240

_STATS_R = 8000
_MLP_R = 4000
_NS = 16
_NW = 32
_CHUNKS = (112000, 208000)

_SC_CHUNK = 128


def _stats_body(x_ref, s_ref, q_ref):
    i = pl.program_id(0)
    xb = x_ref[...]

    @pl.when(i == 0)
    def _():
        s_ref[...] = jnp.zeros_like(s_ref)
        q_ref[...] = jnp.zeros_like(q_ref)

    s_ref[...] += jnp.sum(xb, axis=0, keepdims=True)
    q_ref[...] += jnp.sum(xb * xb, axis=0, keepdims=True)


def _stats(x):
    return pl.pallas_call(
        _stats_body,
        grid=(NTOK // _STATS_R,),
        in_specs=[pl.BlockSpec((_STATS_R, D), lambda i: (i, 0))],
        out_specs=(pl.BlockSpec((1, D), lambda i: (0, 0)),
                   pl.BlockSpec((1, D), lambda i: (0, 0))),
        out_shape=(jax.ShapeDtypeStruct((1, D), jnp.float32),
                   jax.ShapeDtypeStruct((1, D), jnp.float32)),
    )(x)


def _mlp_body(s_ref, q_ref, bnw_ref, bnb_ref, w1_ref, b1_ref, w2_ref, b2_ref,
              x_ref, h_ref):
    mu = s_ref[...] / NTOK
    var = q_ref[...] / NTOK - mu * mu
    scale = bnw_ref[...] * lax.rsqrt(var + 1e-5)
    shift = bnb_ref[...] - mu * scale
    xn = (x_ref[...] * scale + shift).astype(jnp.bfloat16)
    g = lax.dot_general(xn, w1_ref[...], (((1,), (1,)), ((), ())),
                        preferred_element_type=jnp.float32)
    g = jnp.maximum(g + b1_ref[...], 0.0).astype(jnp.bfloat16)
    h = lax.dot_general(g, w2_ref[...], (((1,), (1,)), ((), ())),
                        preferred_element_type=jnp.float32)
    h = h + b2_ref[...]
    h_ref[0, :, :] = h[:, :D]
    h_ref[1, :, :] = h[:, D:]


def _mlp(x, tok0, chtok, s, q, bn1_w, bn1_b, w1, b1, w2, b2):
    full = lambda shape: pl.BlockSpec(shape, lambda i: tuple(0 for _ in shape))
    base = tok0 // _MLP_R
    return pl.pallas_call(
        _mlp_body,
        grid=(chtok // _MLP_R,),
        in_specs=[full((1, D)), full((1, D)), full((1, D)), full((1, D)),
                  full((2 * D, D)), full((1, 2 * D)),
                  full((2 * D, 2 * D)), full((1, 2 * D)),
                  pl.BlockSpec((_MLP_R, D), lambda i: (base + i, 0))],
        out_specs=pl.BlockSpec((2, _MLP_R, D), lambda i: (0, i, 0)),
        out_shape=jax.ShapeDtypeStruct((2, chtok, D), jnp.float32),
    )(s, q, bn1_w.reshape(1, D), bn1_b.reshape(1, D),
      w1.astype(jnp.bfloat16), b1.reshape(1, 2 * D),
      w2.astype(jnp.bfloat16), b2.reshape(1, 2 * D), x)


def _segment_sum(h_rows, idx, init, real_ch, w_ch):
    pad_ch = w_ch * _NW
    npair = w_ch // 2
    mesh = plsc.VectorSubcoreMesh(core_axis_name="c", subcore_axis_name="s")
    cp = pltpu.CompilerParams()
    if "needs_layout_passes" in pltpu.CompilerParams.__dataclass_fields__:
        cp = dataclasses.replace(cp, needs_layout_passes=False)

    @functools.partial(
        pl.kernel,
        mesh=mesh,
        compiler_params=cp,
        out_type=jax.ShapeDtypeStruct((2, NPAD, D), jnp.float32),
        scratch_types=[
            pltpu.VMEM_SHARED((NPAD, D), jnp.float32),
            pltpu.VMEM((_SC_CHUNK, D), jnp.float32),
            pltpu.VMEM((_SC_CHUNK, D), jnp.float32),
            pltpu.VMEM((1, _SC_CHUNK), jnp.int32),
            pltpu.VMEM((1, _SC_CHUNK), jnp.int32),
            pltpu.SemaphoreType.DMA,
            pltpu.SemaphoreType.DMA,
            pltpu.SemaphoreType.DMA,
            pltpu.SemaphoreType.DMA,
        ],
    )
    def scatter_kernel(h_hbm, idx_hbm, init_hbm, out_hbm, acc,
                       buf_a, buf_b, ib_a, ib_b,
                       ra, ia, sa, sb):
        core = lax.axis_index("c")
        sid = lax.axis_index("s")
        wid = sid * 2 + core
        rows_per_tile = NPAD // _NS
        sl = pl.ds(sid * rows_per_tile, rows_per_tile)
        pltpu.sync_copy(init_hbm.at[core].at[sl], acc.at[sl])
        plsc.subcore_barrier()

        def rows_at(c):
            cr = jnp.minimum(c, real_ch - 1)
            return h_hbm.at[pl.ds(cr * _SC_CHUNK, _SC_CHUNK)]

        def start_in(c, buf, ibuf):
            cc = jnp.minimum(c, pad_ch - 1)
            pltpu.async_copy(rows_at(cc), buf, ra)
            pltpu.async_copy(idx_hbm.at[cc], ibuf, ia)

        def wait_in(buf, ibuf):
            pltpu.make_async_copy(rows_at(0), buf, ra).wait()
            pltpu.make_async_copy(idx_hbm.at[0], ibuf, ia).wait()

        def start_scatter(buf, ibuf, sem):
            pltpu.async_copy(buf, acc.at[ibuf.at[0]], sem, add=True)

        def wait_scatter(buf, ibuf, sem):
            pltpu.make_async_copy(buf, acc.at[ibuf.at[0]], sem).wait()

        base = wid * w_ch
        start_in(base, buf_a, ib_a)

        @pl.loop(0, npair)
        def _(jp):
            c = base + 2 * jp
            wait_in(buf_a, ib_a)

            @pl.when(jp > 0)
            def _():
                wait_scatter(buf_b, ib_b, sb)

            start_in(c + 1, buf_b, ib_b)
            start_scatter(buf_a, ib_a, sa)
            wait_in(buf_b, ib_b)
            wait_scatter(buf_a, ib_a, sa)
            start_in(c + 2, buf_a, ib_a)
            start_scatter(buf_b, ib_b, sb)

        wait_scatter(buf_b, ib_b, sb)
        wait_in(buf_a, ib_a)
        plsc.subcore_barrier()
        pltpu.sync_copy(acc.at[sl], out_hbm.at[core].at[sl])

    return scatter_kernel(h_rows, idx, init)


def _node_body(p_ref, bnw_ref, bnb_ref, w3_ref, b3_ref, w4_ref, b4_ref, o_ref):
    nf = p_ref[0, :NNODES, :] + p_ref[1, :NNODES, :]
    mu = jnp.mean(nf, axis=0, keepdims=True)
    var = jnp.mean(nf * nf, axis=0, keepdims=True) - mu * mu
    xn = ((nf - mu) * lax.rsqrt(var + 1e-5) * bnw_ref[...]
          + bnb_ref[...]).astype(jnp.bfloat16)
    g = lax.dot_general(xn, w3_ref[...], (((1,), (1,)), ((), ())),
                        preferred_element_type=jnp.float32)
    g = jnp.maximum(g + b3_ref[...], 0.0).astype(jnp.bfloat16)
    o = lax.dot_general(g, w4_ref[...], (((1,), (1,)), ((), ())),
                        preferred_element_type=jnp.float32)
    o_ref[...] = o + b4_ref[...]


def _node_mlp(partial, bn2_w, bn2_b, w3, b3, w4, b4):
    return pl.pallas_call(
        _node_body,
        out_shape=jax.ShapeDtypeStruct((NNODES, D), jnp.float32),
    )(partial, bn2_w.reshape(1, D), bn2_b.reshape(1, D),
      w3.astype(jnp.bfloat16), b3.reshape(1, D),
      w4.astype(jnp.bfloat16), b4.reshape(1, D))


def kernel(x, node_features, node_batch, token_index, bn1_w, bn1_b, w1, b1,
           w2, b2, bn2_w, bn2_b, w3, b3, w4, b4):
    s, q = _stats(x)
    acc = jnp.zeros((2, NPAD, D), jnp.float32)
    tok0 = 0
    for chtok in _CHUNKS:
        h = _mlp(x, tok0, chtok, s, q, bn1_w, bn1_b, w1, b1, w2, b2)
        ch_rows = 2 * chtok
        real_ch = ch_rows // _SC_CHUNK
        w_ch = ((real_ch + _NW - 1) // _NW + 1) // 2 * 2
        pad_ch = w_ch * _NW
        npadrow = pad_ch * _SC_CHUNK - ch_rows
        pad_idx = NNODES + jnp.arange(npadrow, dtype=jnp.int32) % (NPAD - NNODES)
        idx = jnp.concatenate(
            [token_index[:, tok0:tok0 + chtok].reshape(-1), pad_idx]
        ).reshape(pad_ch, 1, _SC_CHUNK)
        acc = _segment_sum(h.reshape(ch_rows, D), idx, acc, real_ch, w_ch)
        tok0 += chtok
    return _node_mlp(acc, bn2_w, bn2_b, w3, b3, w4, b4)


# speedup vs baseline: 1.1401x; 1.0706x over previous
import dataclasses
import functools

import jax
import jax.numpy as jnp
import numpy as np
from jax import lax
from jax.experimental import pallas as pl
from jax.experimental.pallas import tpu as pltpu
from jax.experimental.pallas import tpu_sc as plsc

D = 128
NTOK = 320000
NNODES = 10000
NPAD = 10112

_STATS_R = 8000
_MLP_R = 4000
_NS = 16
_NW = 32
_CHUNKS = (112000, 208000)

_SC_CHUNK = 128


def _stats_body(x_ref, s_ref, q_ref):
    i = pl.program_id(0)
    xb = x_ref[...]

    @pl.when(i == 0)
    def _():
        s_ref[...] = jnp.zeros_like(s_ref)
        q_ref[...] = jnp.zeros_like(q_ref)

    s_ref[...] += jnp.sum(xb, axis=0, keepdims=True)
    q_ref[...] += jnp.sum(xb * xb, axis=0, keepdims=True)


def _stats(x):
    return pl.pallas_call(
        _stats_body,
        grid=(NTOK // _STATS_R,),
        in_specs=[pl.BlockSpec((_STATS_R, D), lambda i: (i, 0))],
        out_specs=(pl.BlockSpec((1, D), lambda i: (0, 0)),
                   pl.BlockSpec((1, D), lambda i: (0, 0))),
        out_shape=(jax.ShapeDtypeStruct((1, D), jnp.float32),
                   jax.ShapeDtypeStruct((1, D), jnp.float32)),
    )(x)


def _mlp_body(s_ref, q_ref, bnw_ref, bnb_ref, w1_ref, b1_ref, w2_ref, b2_ref,
              x_ref, h_ref):
    mu = s_ref[...] / NTOK
    var = q_ref[...] / NTOK - mu * mu
    scale = bnw_ref[...] * lax.rsqrt(var + 1e-5)
    shift = bnb_ref[...] - mu * scale
    xn = (x_ref[...] * scale + shift).astype(jnp.bfloat16)
    g = lax.dot_general(xn, w1_ref[...], (((1,), (1,)), ((), ())),
                        preferred_element_type=jnp.float32)
    g = jnp.maximum(g + b1_ref[...], 0.0).astype(jnp.bfloat16)
    h = lax.dot_general(g, w2_ref[...], (((1,), (1,)), ((), ())),
                        preferred_element_type=jnp.float32)
    h = h + b2_ref[...]
    h_ref[0, :, :] = h[:, :D]
    h_ref[1, :, :] = h[:, D:]


def _mlp(x, tok0, chtok, s, q, bn1_w, bn1_b, w1, b1, w2, b2):
    full = lambda shape: pl.BlockSpec(shape, lambda i: tuple(0 for _ in shape))
    base = tok0 // _MLP_R
    return pl.pallas_call(
        _mlp_body,
        grid=(chtok // _MLP_R,),
        in_specs=[full((1, D)), full((1, D)), full((1, D)), full((1, D)),
                  full((2 * D, D)), full((1, 2 * D)),
                  full((2 * D, 2 * D)), full((1, 2 * D)),
                  pl.BlockSpec((_MLP_R, D), lambda i: (base + i, 0))],
        out_specs=pl.BlockSpec((2, _MLP_R, D), lambda i: (0, i, 0)),
        out_shape=jax.ShapeDtypeStruct((2, chtok, D), jnp.float32),
    )(s, q, bn1_w.reshape(1, D), bn1_b.reshape(1, D),
      w1.astype(jnp.bfloat16), b1.reshape(1, 2 * D),
      w2.astype(jnp.bfloat16), b2.reshape(1, 2 * D), x)


def _segment_sum(h_rows, idx, init, real_ch, w_ch):
    pad_ch = w_ch * _NW
    ntri = w_ch // 3
    mesh = plsc.VectorSubcoreMesh(core_axis_name="c", subcore_axis_name="s")
    cp = pltpu.CompilerParams()
    if "needs_layout_passes" in pltpu.CompilerParams.__dataclass_fields__:
        cp = dataclasses.replace(cp, needs_layout_passes=False)

    @functools.partial(
        pl.kernel,
        mesh=mesh,
        compiler_params=cp,
        out_type=jax.ShapeDtypeStruct((2, NPAD, D), jnp.float32),
        scratch_types=[
            pltpu.VMEM_SHARED((NPAD, D), jnp.float32),
            pltpu.VMEM((_SC_CHUNK, D), jnp.float32),
            pltpu.VMEM((_SC_CHUNK, D), jnp.float32),
            pltpu.VMEM((_SC_CHUNK, D), jnp.float32),
            pltpu.VMEM((1, _SC_CHUNK), jnp.int32),
            pltpu.VMEM((1, _SC_CHUNK), jnp.int32),
            pltpu.VMEM((1, _SC_CHUNK), jnp.int32),
            pltpu.SemaphoreType.DMA,
            pltpu.SemaphoreType.DMA,
            pltpu.SemaphoreType.DMA,
            pltpu.SemaphoreType.DMA,
            pltpu.SemaphoreType.DMA,
            pltpu.SemaphoreType.DMA,
            pltpu.SemaphoreType.DMA,
            pltpu.SemaphoreType.DMA,
            pltpu.SemaphoreType.DMA,
        ],
    )
    def scatter_kernel(h_hbm, idx_hbm, init_hbm, out_hbm, acc,
                       buf_a, buf_b, buf_c, ib_a, ib_b, ib_c,
                       ra, rb, rc, ja, jb, jc, sa, sb, sc):
        core = lax.axis_index("c")
        sid = lax.axis_index("s")
        wid = sid * 2 + core
        rows_per_tile = NPAD // _NS
        sl = pl.ds(sid * rows_per_tile, rows_per_tile)
        pltpu.sync_copy(init_hbm.at[core].at[sl], acc.at[sl])
        plsc.subcore_barrier()

        def rows_at(c):
            cr = jnp.minimum(c, real_ch - 1)
            return h_hbm.at[pl.ds(cr * _SC_CHUNK, _SC_CHUNK)]

        def start_in(c, buf, ibuf, rs, js):
            cc = jnp.minimum(c, pad_ch - 1)
            pltpu.async_copy(rows_at(cc), buf, rs)
            pltpu.async_copy(idx_hbm.at[cc], ibuf, js)

        def wait_in(buf, ibuf, rs, js):
            pltpu.make_async_copy(rows_at(0), buf, rs).wait()
            pltpu.make_async_copy(idx_hbm.at[0], ibuf, js).wait()

        def start_scatter(buf, ibuf, sem):
            pltpu.async_copy(buf, acc.at[ibuf.at[0]], sem, add=True)

        def wait_scatter(buf, ibuf, sem):
            pltpu.make_async_copy(buf, acc.at[ibuf.at[0]], sem).wait()

        base = wid * w_ch
        start_in(base, buf_a, ib_a, ra, ja)
        start_in(base + 1, buf_b, ib_b, rb, jb)

        @pl.loop(0, ntri)
        def _(jt):
            c = base + 3 * jt
            wait_in(buf_a, ib_a, ra, ja)
            start_scatter(buf_a, ib_a, sa)

            @pl.when(jt > 0)
            def _():
                wait_scatter(buf_c, ib_c, sc)

            start_in(c + 2, buf_c, ib_c, rc, jc)
            wait_in(buf_b, ib_b, rb, jb)
            start_scatter(buf_b, ib_b, sb)
            wait_scatter(buf_a, ib_a, sa)
            start_in(c + 3, buf_a, ib_a, ra, ja)
            wait_in(buf_c, ib_c, rc, jc)
            start_scatter(buf_c, ib_c, sc)
            wait_scatter(buf_b, ib_b, sb)
            start_in(c + 4, buf_b, ib_b, rb, jb)

        wait_in(buf_a, ib_a, ra, ja)
        wait_in(buf_b, ib_b, rb, jb)
        wait_scatter(buf_c, ib_c, sc)
        plsc.subcore_barrier()
        pltpu.sync_copy(acc.at[sl], out_hbm.at[core].at[sl])

    return scatter_kernel(h_rows, idx, init)


def _node_body(p_ref, bnw_ref, bnb_ref, w3_ref, b3_ref, w4_ref, b4_ref, o_ref):
    nf = p_ref[0, :NNODES, :] + p_ref[1, :NNODES, :]
    mu = jnp.mean(nf, axis=0, keepdims=True)
    var = jnp.mean(nf * nf, axis=0, keepdims=True) - mu * mu
    xn = ((nf - mu) * lax.rsqrt(var + 1e-5) * bnw_ref[...]
          + bnb_ref[...]).astype(jnp.bfloat16)
    g = lax.dot_general(xn, w3_ref[...], (((1,), (1,)), ((), ())),
                        preferred_element_type=jnp.float32)
    g = jnp.maximum(g + b3_ref[...], 0.0).astype(jnp.bfloat16)
    o = lax.dot_general(g, w4_ref[...], (((1,), (1,)), ((), ())),
                        preferred_element_type=jnp.float32)
    o_ref[...] = o + b4_ref[...]


def _node_mlp(partial, bn2_w, bn2_b, w3, b3, w4, b4):
    return pl.pallas_call(
        _node_body,
        out_shape=jax.ShapeDtypeStruct((NNODES, D), jnp.float32),
    )(partial, bn2_w.reshape(1, D), bn2_b.reshape(1, D),
      w3.astype(jnp.bfloat16), b3.reshape(1, D),
      w4.astype(jnp.bfloat16), b4.reshape(1, D))


def kernel(x, node_features, node_batch, token_index, bn1_w, bn1_b, w1, b1,
           w2, b2, bn2_w, bn2_b, w3, b3, w4, b4):
    s, q = _stats(x)
    acc = jnp.zeros((2, NPAD, D), jnp.float32)
    tok0 = 0
    for chtok in _CHUNKS:
        h = _mlp(x, tok0, chtok, s, q, bn1_w, bn1_b, w1, b1, w2, b2)
        ch_rows = 2 * chtok
        real_ch = ch_rows // _SC_CHUNK
        w_ch = ((real_ch + _NW - 1) // _NW + 2) // 3 * 3
        pad_ch = w_ch * _NW
        npadrow = pad_ch * _SC_CHUNK - ch_rows
        pad_idx = NNODES + jnp.arange(npadrow, dtype=jnp.int32) % (NPAD - NNODES)
        idx = jnp.concatenate(
            [token_index[:, tok0:tok0 + chtok].reshape(-1), pad_idx]
        ).reshape(pad_ch, 1, _SC_CHUNK)
        acc = _segment_sum(h.reshape(ch_rows, D), idx, acc, real_ch, w_ch)
        tok0 += chtok
    return _node_mlp(acc, bn2_w, bn2_b, w3, b3, w4, b4)
